# baseline (speedup 1.0000x reference)
import functools

import jax
import jax.numpy as jnp
from jax import lax
from jax.experimental import pallas as pl
from jax.experimental.pallas import tpu as pltpu
from jax.experimental.pallas import tpu_sc as plsc

_INTERPRET = False
_USE_SC = True

EMB = 128
BINS = 512
NP_NODES = 10240
N_WORKERS = 32
CHUNK = 40


def _ln(xv, g, b, eps=1e-5):
    mu = jnp.mean(xv, axis=-1, keepdims=True)
    xc = xv - mu
    var = jnp.mean(xc * xc, axis=-1, keepdims=True)
    return xc * lax.rsqrt(var + eps) * g + b


def _dot(a, b):
    return jnp.dot(a, b, preferred_element_type=jnp.float32)


def _expansion_t(f, vmin, delta):
    g = 1.0 / delta
    c = vmin + delta * lax.broadcasted_iota(jnp.int32, (BINS, 1), 0).astype(jnp.float32)
    d = f - c
    return jnp.exp(d * (d * (-g)))


def _dot_t(z, w):
    return jax.lax.dot_general(z, w, (((0,), (0,)), ((), ())),
                               preferred_element_type=jnp.float32)


def _row_spec(t, w):
    return pl.BlockSpec((t, w), lambda i: (i, 0))


def _full_spec(shape):
    nd = len(shape)
    return pl.BlockSpec(shape, lambda i: (0,) * nd)


def _rbf_edge_body(ea_ref, w_ref, b_ref, o_ref, *, vmin, vmax):
    a = ea_ref[...]
    n2 = jnp.sum(a * a, axis=0, keepdims=True)
    f = -0.75 / jnp.sqrt(n2)
    delta = (vmax - vmin) / (BINS - 1)
    z = _expansion_t(f, vmin, delta)
    o_ref[...] = jax.nn.softplus(_dot_t(z, w_ref[...]) + b_ref[...])


def _rbf_edge(ea_t, w, b, tile):
    e = ea_t.shape[1]
    return pl.pallas_call(
        functools.partial(_rbf_edge_body, vmin=-4.0, vmax=0.0),
        grid=(e // tile,),
        in_specs=[pl.BlockSpec((3, tile), lambda i: (0, i)),
                  _full_spec((BINS, EMB)), _full_spec((1, EMB))],
        out_specs=_row_spec(tile, EMB),
        out_shape=jax.ShapeDtypeStruct((e, EMB), jnp.float32),
        interpret=_INTERPRET,
    )(ea_t, w, b)


def _rbf_len_body(nei_ref, wl_ref, bl_ref, l0_ref, l1_ref, l2_ref):
    nei = nei_ref[...]
    wl = wl_ref[...]
    louts = (l0_ref, l1_ref, l2_ref)
    dl = 4.0 / (BINS - 1)
    for s in range(3):
        r1 = nei[3 * s:3 * s + 3, :]
        n1 = jnp.sqrt(jnp.sum(r1 * r1, axis=0, keepdims=True))
        nlen = -0.75 / n1
        zl = _expansion_t(nlen, -4.0, dl)
        louts[s][...] = jax.nn.softplus(_dot_t(zl, wl) + bl_ref[...])


def _rbf_len(nei_t, wl, bl, tile):
    e = nei_t.shape[1]
    return pl.pallas_call(
        _rbf_len_body,
        grid=(e // tile,),
        in_specs=[pl.BlockSpec((9, tile), lambda i: (0, i)),
                  _full_spec((BINS, EMB)), _full_spec((1, EMB))],
        out_specs=[_row_spec(tile, EMB)] * 3,
        out_shape=[jax.ShapeDtypeStruct((e, EMB), jnp.float32)] * 3,
        interpret=_INTERPRET,
    )(nei_t, wl, bl)


def _rbf_ang_body(nei_ref, ea_ref, wa_ref, ba_ref, a0_ref, a1_ref, a2_ref):
    nei = nei_ref[...]
    r2 = ea_ref[...]
    n2 = jnp.sqrt(jnp.sum(r2 * r2, axis=0, keepdims=True))
    wa = wa_ref[...]
    aouts = (a0_ref, a1_ref, a2_ref)
    da = 2.0 / (BINS - 1)
    for s in range(3):
        r1 = nei[3 * s:3 * s + 3, :]
        dot = jnp.sum(r1 * r2, axis=0, keepdims=True)
        n1 = jnp.sqrt(jnp.sum(r1 * r1, axis=0, keepdims=True))
        nprod = n1 * n2
        cos = jnp.clip(dot / (nprod + 1e-8), -1.0, 1.0)
        cos = jnp.where(nprod == 0, 1.0, cos)
        za = _expansion_t(cos, -1.0, da)
        aouts[s][...] = jax.nn.softplus(_dot_t(za, wa) + ba_ref[...])


def _rbf_ang(nei_t, ea_t, wa, ba, tile):
    e = nei_t.shape[1]
    return pl.pallas_call(
        _rbf_ang_body,
        grid=(e // tile,),
        in_specs=[pl.BlockSpec((9, tile), lambda i: (0, i)),
                  pl.BlockSpec((3, tile), lambda i: (0, i)),
                  _full_spec((BINS, EMB)), _full_spec((1, EMB))],
        out_specs=[_row_spec(tile, EMB)] * 3,
        out_shape=[jax.ShapeDtypeStruct((e, EMB), jnp.float32)] * 3,
        interpret=_INTERPRET,
    )(nei_t, ea_t, wa, ba)


def _matmul_body(x_ref, w_ref, b_ref, o_ref):
    o_ref[...] = _dot(x_ref[...], w_ref[...]) + b_ref[...]


def _node_matmul(x, w, b, tile):
    n = x.shape[0]
    din, dout = w.shape
    return pl.pallas_call(
        _matmul_body,
        grid=(n // tile,),
        in_specs=[_row_spec(tile, din), _full_spec((din, dout)), _full_spec((1, dout))],
        out_specs=_row_spec(tile, dout),
        out_shape=jax.ShapeDtypeStruct((n, dout), jnp.float32),
        interpret=_INTERPRET,
    )(x, w, b)


def _qkv_body(x_ref, wq_ref, bq_ref, wkv_ref, bkv_ref, q_ref, kv_ref):
    xv = x_ref[...]
    q_ref[...] = _dot(xv, wq_ref[...]) + bq_ref[...]
    kv_ref[...] = _dot(xv, wkv_ref[...]) + bkv_ref[...]


def _qkv(node, wq, bq, wkv, bkv, tile):
    n = node.shape[0]
    return pl.pallas_call(
        _qkv_body,
        grid=(n // tile,),
        in_specs=[_row_spec(tile, EMB), _full_spec((EMB, EMB)), _full_spec((1, EMB)),
                  _full_spec((EMB, 2 * EMB)), _full_spec((1, 2 * EMB))],
        out_specs=[_row_spec(tile, EMB), _row_spec(tile, 2 * EMB)],
        out_shape=[jax.ShapeDtypeStruct((n, EMB), jnp.float32),
                   jax.ShapeDtypeStruct((n, 2 * EMB), jnp.float32)],
        interpret=_INTERPRET,
    )(node, wq, bq, wkv, bkv)


def _ek_em_body(ef_ref, wek_ref, bek_ref, wem_ref, bem_ref, ek_ref, em_ref):
    ef = ef_ref[...]
    ek_ref[...] = _dot(ef, wek_ref[...]) + bek_ref[...]
    em_ref[...] = _dot(ef, wem_ref[...]) + bem_ref[...]


def _ek_em(ef, wek, bek, wem, bem, tile):
    e = ef.shape[0]
    return pl.pallas_call(
        _ek_em_body,
        grid=(e // tile,),
        in_specs=[_row_spec(tile, EMB)] + [_full_spec((EMB, EMB)), _full_spec((1, EMB))] * 2,
        out_specs=[_row_spec(tile, EMB)] * 2,
        out_shape=[jax.ShapeDtypeStruct((e, EMB), jnp.float32)] * 2,
        interpret=_INTERPRET,
    )(ef, wek, bek, wem, bem)


def _conv_msg_body(qd_ref, kvs_ref, ef_ref,
                   wek_ref, bek_ref, wem_ref, bem_ref,
                   w1k_ref, w2k_ref, b2k_ref,
                   w1m_ref, w2m_ref, b2m_ref, wmsg_ref, bmsg_ref,
                   attg_ref, attb_ref, msgg_ref, msgb_ref, o_ref):
    qd = qd_ref[...]
    kvs = kvs_ref[...]
    ef = ef_ref[...]
    ks = kvs[:, :EMB]
    vs = kvs[:, EMB:]
    ek = _dot(ef, wek_ref[...]) + bek_ref[...]
    em = _dot(ef, wem_ref[...]) + bem_ref[...]
    hk = _dot(jnp.concatenate([ks, qd], axis=-1), w1k_ref[...]) + ek
    key = _dot(jax.nn.silu(hk), w2k_ref[...]) + b2k_ref[...]
    alpha = qd * key * (1.0 / jnp.sqrt(jnp.float32(EMB)))
    hm = _dot(jnp.concatenate([vs, qd], axis=-1), w1m_ref[...]) + em
    msg = _dot(jax.nn.silu(hm), w2m_ref[...]) + b2m_ref[...]
    msg = msg * jax.nn.sigmoid(_ln(alpha, attg_ref[...], attb_ref[...]))
    msg = _ln(_dot(msg, wmsg_ref[...]) + bmsg_ref[...], msgg_ref[...], msgb_ref[...])
    o_ref[...] = msg


def _conv_msg(qd, kvs, ef, wts, tile):
    e = qd.shape[0]
    specs = [_row_spec(tile, EMB), _row_spec(tile, 2 * EMB), _row_spec(tile, EMB)]
    specs += [_full_spec(w.shape) for w in wts]
    return pl.pallas_call(
        _conv_msg_body,
        grid=(e // tile,),
        in_specs=specs,
        out_specs=_row_spec(tile, EMB),
        out_shape=jax.ShapeDtypeStruct((e, EMB), jnp.float32),
        interpret=_INTERPRET,
    )(qd, kvs, ef, *wts)


def _node_upd_body(node_ref, p0_ref, p1_ref, wcc_ref, bcc_ref, g_ref, b_ref, o_ref):
    agg = p0_ref[...] + p1_ref[...]
    out = (_dot(agg, wcc_ref[...]) + bcc_ref[...]) * g_ref[...] + b_ref[...]
    o_ref[...] = jax.nn.softplus(node_ref[...] + out)


def _node_update(node, p0, p1, wcc, bcc, g, b, tile):
    n = node.shape[0]
    return pl.pallas_call(
        _node_upd_body,
        grid=(n // tile,),
        in_specs=[_row_spec(tile, EMB)] * 3 + [_full_spec((EMB, EMB))] + [_full_spec((1, EMB))] * 3,
        out_specs=_row_spec(tile, EMB),
        out_shape=jax.ShapeDtypeStruct((n, EMB), jnp.float32),
        interpret=_INTERPRET,
    )(node, p0, p1, wcc, bcc, g, b)


def _edge_upd_body(ef_ref, nei_ref, a0_ref, a1_ref, a2_ref,
                   wrbfl_ref, brbfl_ref,
                   wq_ref, bq_ref, wk_ref, bk_ref, wv_ref, bv_ref,
                   wlen_ref, lembb_ref,
                   wke0_ref, bke0_ref, wke1_ref, bke1_ref, wke2_ref, bke2_ref,
                   wve0_ref, bve0_ref, wve1_ref, bve1_ref, wve2_ref, bve2_ref,
                   we_ref, be_ref, w1k_ref, b1k_ref, w2k_ref, b2k_ref,
                   w1m_ref, b1m_ref, w2m_ref, b2m_ref, wcc_ref, bcc_ref,
                   attg_ref, attb_ref, bng_ref, bnb_ref, o_ref):
    ed = ef_ref[...]
    nei = nei_ref[...]
    w1k = w1k_ref[...]
    w1m = w1m_ref[...]
    wrbfl = wrbfl_ref[...]
    qx = _dot(ed, wq_ref[...]) + bq_ref[...]
    kx1 = _dot(_dot(ed, wk_ref[...]) + bk_ref[...], w1k[:EMB])
    vx1 = _dot(_dot(ed, wv_ref[...]) + bv_ref[...], w1m[:EMB])
    rsq = 1.0 / jnp.sqrt(jnp.float32(EMB))
    arefs = (a0_ref, a1_ref, a2_ref)
    kerefs = ((wke0_ref, bke0_ref), (wke1_ref, bke1_ref), (wke2_ref, bke2_ref))
    verefs = ((wve0_ref, bve0_ref), (wve1_ref, bve1_ref), (wve2_ref, bve2_ref))
    acc = jnp.zeros_like(ed)
    dl = 4.0 / (BINS - 1)
    for s in range(3):
        r1 = nei[3 * s:3 * s + 3, :]
        n1 = jnp.sqrt(jnp.sum(r1 * r1, axis=0, keepdims=True))
        zl = _expansion_t(-0.75 / n1, -4.0, dl)
        lf = jax.nn.softplus(_dot_t(zl, wrbfl) + brbfl_ref[...])
        nl = jax.nn.silu(_dot(lf, wlen_ref[...]) + lembb_ref[s:s + 1, :])
        ky = _dot(nl, kerefs[s][0][...]) + kerefs[s][1][...]
        vy = _dot(nl, verefs[s][0][...]) + verefs[s][1][...]
        exy = _dot(arefs[s][...], we_ref[...]) + be_ref[...]
        hk = kx1 + _dot(ky, w1k[EMB:2 * EMB]) + _dot(exy, w1k[2 * EMB:]) + b1k_ref[...]
        kk = _dot(jax.nn.silu(hk), w2k_ref[...]) + b2k_ref[...]
        alpha = qx * kk * rsq
        hm = vx1 + _dot(vy, w1m[EMB:2 * EMB]) + _dot(exy, w1m[2 * EMB:]) + b1m_ref[...]
        oo = _dot(jax.nn.silu(hm), w2m_ref[...]) + b2m_ref[...]
        acc += oo * jax.nn.sigmoid(_ln(alpha, attg_ref[...], attb_ref[...]))
    res = _dot(acc, wcc_ref[...]) * (1.0 / 3.0) + bcc_ref[...]
    o_ref[...] = jax.nn.softplus(ed + res * bng_ref[...] + bnb_ref[...])


def _edge_update(ef, nei_t, angs, wts, tile):
    e = ef.shape[0]
    nt = e // tile
    specs = [_row_spec(tile, EMB), pl.BlockSpec((9, tile), lambda i: (0, i))]
    specs += [_row_spec(tile, EMB)] * 3
    specs += [_full_spec(w.shape) for w in wts]
    return pl.pallas_call(
        _edge_upd_body,
        grid=(nt,),
        in_specs=specs,
        out_specs=_row_spec(tile, EMB),
        out_shape=jax.ShapeDtypeStruct((e, EMB), jnp.float32),
        interpret=_INTERPRET,
    )(ef, nei_t, angs[0], angs[1], angs[2], *wts)


def _pool_body(node_ref, b2_ref, ps_ref, cnt_ref, *, nb):
    i = pl.program_id(0)

    @pl.when(i == 0)
    def _():
        ps_ref[...] = jnp.zeros_like(ps_ref)
        cnt_ref[...] = jnp.zeros_like(cnt_ref)

    bv = b2_ref[...]
    t = bv.shape[0]
    oh = (bv == lax.broadcasted_iota(jnp.int32, (t, nb), 1)).astype(jnp.float32)
    dn = (((0,), (0,)), ((), ()))
    ps_ref[...] += lax.dot_general(oh, node_ref[...], dn,
                                   preferred_element_type=jnp.float32)
    cnt_ref[...] += lax.dot_general(oh, jnp.ones((t, EMB), jnp.float32), dn,
                                    preferred_element_type=jnp.float32)


def _pool(node, batch2d, nb, tile):
    n = node.shape[0]
    return pl.pallas_call(
        functools.partial(_pool_body, nb=nb),
        grid=(n // tile,),
        in_specs=[_row_spec(tile, EMB), pl.BlockSpec((tile, 1), lambda i: (i, 0))],
        out_specs=[_full_spec((nb, EMB)), _full_spec((nb, EMB))],
        out_shape=[jax.ShapeDtypeStruct((nb, EMB), jnp.float32),
                   jax.ShapeDtypeStruct((nb, EMB), jnp.float32)],
        interpret=_INTERPRET,
    )(node, batch2d)


def _polar_cols(c):
    for _ in range(12):
        cof = [c[4] * c[8] - c[5] * c[7], c[5] * c[6] - c[3] * c[8], c[3] * c[7] - c[4] * c[6],
               c[2] * c[7] - c[1] * c[8], c[0] * c[8] - c[2] * c[6], c[1] * c[6] - c[0] * c[7],
               c[1] * c[5] - c[2] * c[4], c[2] * c[3] - c[0] * c[5], c[0] * c[4] - c[1] * c[3]]
        det = c[0] * cof[0] + c[1] * cof[1] + c[2] * cof[2]
        adet = jnp.maximum(jnp.abs(det), 1e-30)
        mu = jnp.exp(jnp.log(adet) * (-1.0 / 3.0))
        inv_md = 1.0 / (mu * det)
        c = [0.5 * (mu * c[k] + cof[k] * inv_md) for k in range(9)]
    return c


def _head_body(ps_ref, cnt_ref, at_ref, w1_ref, b1_ref, w2_ref, b2_ref, o_ref):
    pooled = ps_ref[...] / jnp.maximum(cnt_ref[...], 1.0)
    h0 = _dot(pooled, w1_ref[...]) + b1_ref[...]
    h = jnp.where(h0 > 0, h0, jnp.exp(jnp.minimum(h0, 0.0)) - 1.0)
    o = _dot(h, w2_ref[...]) + b2_ref[...]
    a = at_ref[...]
    r = _polar_cols([a[:, k:k + 1] for k in range(9)])
    m = [o[:, k:k + 1] for k in range(9)]
    p = [r[3 * i + 0] * m[3 * 0 + j] + r[3 * i + 1] * m[3 * 1 + j] + r[3 * i + 2] * m[3 * 2 + j]
         for i in range(3) for j in range(3)]
    q = [p[3 * i + 0] * r[3 * j + 0] + p[3 * i + 1] * r[3 * j + 1] + p[3 * i + 2] * r[3 * j + 2]
         for i in range(3) for j in range(3)]
    o_ref[...] = jnp.concatenate(q + [jnp.zeros_like(q[0])] * 7, axis=-1)


def _head(ps, cnt, at16, w1, b1, w2, b2):
    nb = ps.shape[0]
    return pl.pallas_call(
        _head_body,
        grid=(1,),
        in_specs=[_full_spec((nb, EMB)), _full_spec((nb, EMB)), _full_spec((nb, 16)),
                  _full_spec((EMB, EMB)), _full_spec((1, EMB)),
                  _full_spec((EMB, 16)), _full_spec((1, 16))],
        out_specs=_full_spec((nb, 16)),
        out_shape=jax.ShapeDtypeStruct((nb, 16), jnp.float32),
        interpret=_INTERPRET,
    )(ps, cnt, at16, w1, b1, w2, b2)


def _sc_gather_pair(q_tab, kv_tab, idx_dst3, idx_src3, e):
    per_w = e // N_WORKERS
    nchunk = per_w // CHUNK
    mesh = plsc.VectorSubcoreMesh(core_axis_name="c", subcore_axis_name="s")

    kd = 5
    ngroups = nchunk // kd

    @functools.partial(
        pl.kernel, mesh=mesh,
        out_type=[jax.ShapeDtypeStruct((e, EMB), jnp.float32),
                  jax.ShapeDtypeStruct((e, 2 * EMB), jnp.float32)],
        scratch_types=[pltpu.VMEM((nchunk, CHUNK), jnp.int32),
                       pltpu.VMEM((nchunk, CHUNK), jnp.int32),
                       pltpu.VMEM((kd, CHUNK, EMB), jnp.float32),
                       pltpu.VMEM((kd, CHUNK, 2 * EMB), jnp.float32),
                       pltpu.SemaphoreType.DMA,
                       pltpu.SemaphoreType.DMA,
                       pltpu.SemaphoreType.DMA,
                       pltpu.SemaphoreType.DMA],
    )
    def k(qt, kvt, idxd, idxs, qd_out, kvs_out, idxd_v, idxs_v, qbuf, kvbuf,
          sgq, sgk, soq, sok):
        wid = lax.axis_index("s") * 2 + lax.axis_index("c")
        base = wid * per_w
        pltpu.sync_copy(idxd.at[wid], idxd_v)
        pltpu.sync_copy(idxs.at[wid], idxs_v)

        def body(g, carry):
            j0 = g * kd
            hq = [pltpu.async_copy(qt.at[idxd_v.at[j0 + b]], qbuf.at[b], sgq)
                  for b in range(kd)]
            hk = [pltpu.async_copy(kvt.at[idxs_v.at[j0 + b]], kvbuf.at[b], sgk)
                  for b in range(kd)]
            for h in hq:
                h.wait()
            oq = [pltpu.async_copy(qbuf.at[b],
                                   qd_out.at[pl.ds(base + (j0 + b) * CHUNK, CHUNK)], soq)
                  for b in range(kd)]
            for h in hk:
                h.wait()
            ok = [pltpu.async_copy(kvbuf.at[b],
                                   kvs_out.at[pl.ds(base + (j0 + b) * CHUNK, CHUNK)], sok)
                  for b in range(kd)]
            for h in oq + ok:
                h.wait()
            return carry

        lax.fori_loop(0, ngroups, body, 0)

    return k(q_tab, kv_tab, idx_dst3, idx_src3)


def _sc_scatter_add(msg, idx_dst3, zeros_tab, e):
    per_w = e // N_WORKERS
    nchunk = per_w // CHUNK
    np_ = zeros_tab.shape[0]
    rows_per_sub = np_ // 16
    mesh = plsc.VectorSubcoreMesh(core_axis_name="c", subcore_axis_name="s")

    kd = 5
    ngroups = nchunk // kd

    @functools.partial(
        pl.kernel, mesh=mesh,
        out_type=jax.ShapeDtypeStruct((2, np_, EMB), jnp.float32),
        scratch_types=[pltpu.VMEM_SHARED((np_, EMB), jnp.float32),
                       pltpu.VMEM((nchunk, CHUNK), jnp.int32),
                       pltpu.VMEM((kd * CHUNK, EMB), jnp.float32),
                       pltpu.SemaphoreType.DMA],
    )
    def k(msg_h, idxd, zeros_h, out, shared, idx_v, mbuf, ssa):
        cid = lax.axis_index("c")
        sid = lax.axis_index("s")
        wid = sid * 2 + cid
        base = wid * per_w
        row0 = sid * rows_per_sub
        pltpu.sync_copy(zeros_h.at[pl.ds(row0, rows_per_sub)],
                        shared.at[pl.ds(row0, rows_per_sub)])
        plsc.subcore_barrier()
        pltpu.sync_copy(idxd.at[wid], idx_v)

        def body(g, carry):
            j0 = g * kd
            pltpu.sync_copy(msg_h.at[pl.ds(base + j0 * CHUNK, kd * CHUNK)], mbuf)
            hs = [pltpu.async_copy(mbuf.at[pl.ds(b * CHUNK, CHUNK)],
                                   shared.at[idx_v.at[j0 + b]], ssa, add=True)
                  for b in range(kd)]
            for h in hs:
                h.wait()
            return carry

        lax.fori_loop(0, ngroups, body, 0)
        plsc.subcore_barrier()
        pltpu.sync_copy(shared.at[pl.ds(row0, rows_per_sub)],
                        out.at[cid, pl.ds(row0, rows_per_sub)])

    return k(msg, idx_dst3, zeros_tab)


def kernel(x, edge_attr, edge_nei, pos, cell, edge_index, batch, params):
    p = params
    n = x.shape[0]
    e = edge_attr.shape[0]
    nb = cell.shape[0]
    np_ = -(-n // 2048) * 2048
    t_e = 1600 if e % 1600 == 0 else 400
    t_r = 1280 if e % 1280 == 0 else 128
    t_n = 512
    f32 = jnp.float32

    xp = jnp.zeros((np_, EMB), f32).at[:n, :x.shape[1]].set(x)
    batch_p = jnp.concatenate([batch.astype(jnp.int32),
                               jnp.full((np_ - n,), nb, jnp.int32)])
    batch2d = batch_p.reshape(np_, 1)
    src = edge_index[0].astype(jnp.int32)
    dst = edge_index[1].astype(jnp.int32)
    nei9 = edge_nei.reshape(e, 9)

    def b1(arr):
        return arr.reshape(1, -1)

    w_atom = jnp.zeros((EMB, EMB), f32).at[:p['atom_emb']['w'].shape[0]].set(p['atom_emb']['w'])
    node = _node_matmul(xp, w_atom, b1(p['atom_emb']['b']), t_n)

    if _USE_SC:
        idx_dst3 = dst.reshape(N_WORKERS, -1, CHUNK)
        idx_src3 = src.reshape(N_WORKERS, -1, CHUNK)
        zeros_tab = jnp.zeros((np_, EMB), f32)

    def gather_stage(node, cp):
        wkv = jnp.concatenate([cp['k']['w'], cp['v']['w']], axis=1)
        bkv = jnp.concatenate([cp['k']['b'], cp['v']['b']]).reshape(1, -1)
        q_tab, kv_tab = _qkv(node, cp['q']['w'], b1(cp['q']['b']), wkv, bkv, t_n)
        if _USE_SC:
            return _sc_gather_pair(q_tab, kv_tab, idx_dst3, idx_src3, e)
        return q_tab[dst], kv_tab[src]

    def msg_stage(qd, kvs, ef, cp):
        w1k = cp['ku']['l1']['w']
        w1m = cp['mu']['l1']['w']
        wek = cp['e']['w'] @ w1k[2 * EMB:]
        bek = (cp['e']['b'] @ w1k[2 * EMB:] + cp['ku']['l1']['b']).reshape(1, -1)
        wem = cp['e']['w'] @ w1m[2 * EMB:]
        bem = (cp['e']['b'] @ w1m[2 * EMB:] + cp['mu']['l1']['b']).reshape(1, -1)
        wts = [wek, bek, wem, bem,
               w1k[:2 * EMB],
               cp['ku']['l2']['w'], b1(cp['ku']['l2']['b']),
               w1m[:2 * EMB],
               cp['mu']['l2']['w'], b1(cp['mu']['l2']['b']),
               cp['msg']['w'], b1(cp['msg']['b']),
               b1(cp['att_ln_g']), b1(cp['att_ln_b']),
               b1(cp['msg_ln_g']), b1(cp['msg_ln_b'])]
        msg = _conv_msg(qd, kvs, ef, wts, t_e)
        if _USE_SC:
            part = _sc_scatter_add(msg, idx_dst3, zeros_tab, e)
            return part[0], part[1]
        return jax.ops.segment_sum(msg, dst, num_segments=np_), None

    def upd_stage(node, p0, p1, cp):
        if p1 is None:
            p1 = jnp.zeros_like(p0)
        return _node_update(node, p0, p1, cp['cc']['w'], b1(cp['cc']['b']),
                            b1(cp['bn_g']), b1(cp['bn_b']), t_n)

    def conv(node, ef, cp):
        qd, kvs = gather_stage(node, cp)
        p0, p1 = msg_stage(qd, kvs, ef, cp)
        return upd_stage(node, p0, p1, cp)

    ea_t = edge_attr.T
    nei_t = nei9.T
    cp0 = p['att0']
    qd0, kvs0 = gather_stage(node, cp0)
    ef = _rbf_edge(ea_t, p['rbf_lin']['w'], b1(p['rbf_lin']['b']), t_r)
    pt0, pt1 = msg_stage(qd0, kvs0, ef, cp0)
    angs = _rbf_ang(nei_t, ea_t, p['rbf_angle_lin']['w'],
                    b1(p['rbf_angle_lin']['b']), t_r)
    if pt1 is not None:
        bar = jax.lax.optimization_barrier((pt0, pt1) + tuple(angs))
        pt0, pt1 = bar[0], bar[1]
        angs = bar[2:5]

    ep = p['edge_upd']
    wlen_a = ep['len']['w'][:EMB]
    lembb = ep['lemb'] @ ep['len']['w'][EMB:] + ep['len']['b']
    lembb8 = jnp.zeros((8, EMB), f32).at[:3].set(lembb)
    ewts = [p['rbf_lin']['w'], b1(p['rbf_lin']['b']),
            ep['q']['w'], b1(ep['q']['b']), ep['k']['w'], b1(ep['k']['b']),
            ep['v']['w'], b1(ep['v']['b']), wlen_a, lembb8,
            ep['ke1']['w'], b1(ep['ke1']['b']), ep['ke2']['w'], b1(ep['ke2']['b']),
            ep['ke3']['w'], b1(ep['ke3']['b']),
            ep['ve1']['w'], b1(ep['ve1']['b']), ep['ve2']['w'], b1(ep['ve2']['b']),
            ep['ve3']['w'], b1(ep['ve3']['b']),
            ep['e']['w'], b1(ep['e']['b']),
            ep['ku']['l1']['w'], b1(ep['ku']['l1']['b']),
            ep['ku']['l2']['w'], b1(ep['ku']['l2']['b']),
            ep['mu']['l1']['w'], b1(ep['mu']['l1']['b']),
            ep['mu']['l2']['w'], b1(ep['mu']['l2']['b']),
            ep['cc']['w'], b1(ep['cc']['b']),
            b1(ep['att_ln_g']), b1(ep['att_ln_b']),
            b1(ep['bn_g']), b1(ep['bn_b'])]
    node = upd_stage(node, pt0, pt1, cp0)
    cp1 = p['att1']
    qd1, kvs1 = gather_stage(node, cp1)
    ef = _edge_update(ef, nei_t, angs, ewts, t_r)
    pt0, pt1 = msg_stage(qd1, kvs1, ef, cp1)
    node = upd_stage(node, pt0, pt1, cp1)
    node = conv(node, ef, p['att2'])
    node = conv(node, ef, p['att3'])

    ps, cnt = _pool(node, batch2d, nb, t_n)
    at9 = jnp.swapaxes(cell, -2, -1).reshape(nb, 9)
    at16 = jnp.zeros((nb, 16), f32).at[:, :9].set(at9)
    w2 = jnp.zeros((EMB, 16), f32).at[:, :9].set(p['fc2']['w'])
    b2 = jnp.zeros((1, 16), f32).at[0, :9].set(p['fc2']['b'])
    outf = _head(ps, cnt, at16, p['fc1']['w'], b1(p['fc1']['b']), w2, b2)
    return outf[:, :9].reshape(nb, 3, 3)


# speedup vs baseline: 1.0213x; 1.0213x over previous
import functools

import jax
import jax.numpy as jnp
from jax import lax
from jax.experimental import pallas as pl
from jax.experimental.pallas import tpu as pltpu
from jax.experimental.pallas import tpu_sc as plsc

_INTERPRET = False
_USE_SC = True

EMB = 128
BINS = 512
NP_NODES = 10240
N_WORKERS = 32
CHUNK = 40


def _ln(xv, g, b, eps=1e-5):
    mu = jnp.mean(xv, axis=-1, keepdims=True)
    xc = xv - mu
    var = jnp.mean(xc * xc, axis=-1, keepdims=True)
    return xc * lax.rsqrt(var + eps) * g + b


def _dot(a, b):
    return jnp.dot(a, b, preferred_element_type=jnp.float32)


def _expansion_t(f, vmin, delta):
    g = 1.0 / delta
    c = vmin + delta * lax.broadcasted_iota(jnp.int32, (BINS, 1), 0).astype(jnp.float32)
    d = f - c
    return jnp.exp(d * (d * (-g)))


def _dot_t(z, w):
    return jax.lax.dot_general(z, w, (((0,), (0,)), ((), ())),
                               preferred_element_type=jnp.float32)


def _row_spec(t, w):
    return pl.BlockSpec((t, w), lambda i: (i, 0))


def _full_spec(shape):
    nd = len(shape)
    return pl.BlockSpec(shape, lambda i: (0,) * nd)


def _rbf_edge_body(ea_ref, w_ref, b_ref, o_ref, *, vmin, vmax):
    a = ea_ref[...]
    n2 = jnp.sum(a * a, axis=0, keepdims=True)
    f = -0.75 / jnp.sqrt(n2)
    delta = (vmax - vmin) / (BINS - 1)
    z = _expansion_t(f, vmin, delta)
    o_ref[...] = jax.nn.softplus(_dot_t(z, w_ref[...]) + b_ref[...])


def _rbf_edge(ea_t, w, b, tile):
    e = ea_t.shape[1]
    return pl.pallas_call(
        functools.partial(_rbf_edge_body, vmin=-4.0, vmax=0.0),
        grid=(e // tile,),
        in_specs=[pl.BlockSpec((3, tile), lambda i: (0, i)),
                  _full_spec((BINS, EMB)), _full_spec((1, EMB))],
        out_specs=_row_spec(tile, EMB),
        out_shape=jax.ShapeDtypeStruct((e, EMB), jnp.float32),
        interpret=_INTERPRET,
    )(ea_t, w, b)


def _rbf_len_body(nei_ref, wl_ref, bl_ref, l0_ref, l1_ref, l2_ref):
    nei = nei_ref[...]
    wl = wl_ref[...]
    louts = (l0_ref, l1_ref, l2_ref)
    dl = 4.0 / (BINS - 1)
    for s in range(3):
        r1 = nei[3 * s:3 * s + 3, :]
        n1 = jnp.sqrt(jnp.sum(r1 * r1, axis=0, keepdims=True))
        nlen = -0.75 / n1
        zl = _expansion_t(nlen, -4.0, dl)
        louts[s][...] = jax.nn.softplus(_dot_t(zl, wl) + bl_ref[...])


def _rbf_len(nei_t, wl, bl, tile):
    e = nei_t.shape[1]
    return pl.pallas_call(
        _rbf_len_body,
        grid=(e // tile,),
        in_specs=[pl.BlockSpec((9, tile), lambda i: (0, i)),
                  _full_spec((BINS, EMB)), _full_spec((1, EMB))],
        out_specs=[_row_spec(tile, EMB)] * 3,
        out_shape=[jax.ShapeDtypeStruct((e, EMB), jnp.float32)] * 3,
        interpret=_INTERPRET,
    )(nei_t, wl, bl)


def _rbf_ang_body(nei_ref, ea_ref, wa_ref, ba_ref, a0_ref, a1_ref, a2_ref):
    nei = nei_ref[...]
    r2 = ea_ref[...]
    n2 = jnp.sqrt(jnp.sum(r2 * r2, axis=0, keepdims=True))
    wa = wa_ref[...]
    aouts = (a0_ref, a1_ref, a2_ref)
    da = 2.0 / (BINS - 1)
    for s in range(3):
        r1 = nei[3 * s:3 * s + 3, :]
        dot = jnp.sum(r1 * r2, axis=0, keepdims=True)
        n1 = jnp.sqrt(jnp.sum(r1 * r1, axis=0, keepdims=True))
        nprod = n1 * n2
        cos = jnp.clip(dot / (nprod + 1e-8), -1.0, 1.0)
        cos = jnp.where(nprod == 0, 1.0, cos)
        za = _expansion_t(cos, -1.0, da)
        aouts[s][...] = jax.nn.softplus(_dot_t(za, wa) + ba_ref[...])


def _rbf_ang(nei_t, ea_t, wa, ba, tile):
    e = nei_t.shape[1]
    return pl.pallas_call(
        _rbf_ang_body,
        grid=(e // tile,),
        in_specs=[pl.BlockSpec((9, tile), lambda i: (0, i)),
                  pl.BlockSpec((3, tile), lambda i: (0, i)),
                  _full_spec((BINS, EMB)), _full_spec((1, EMB))],
        out_specs=[_row_spec(tile, EMB)] * 3,
        out_shape=[jax.ShapeDtypeStruct((e, EMB), jnp.float32)] * 3,
        interpret=_INTERPRET,
    )(nei_t, ea_t, wa, ba)


def _matmul_body(x_ref, w_ref, b_ref, o_ref):
    o_ref[...] = _dot(x_ref[...], w_ref[...]) + b_ref[...]


def _node_matmul(x, w, b, tile):
    n = x.shape[0]
    din, dout = w.shape
    return pl.pallas_call(
        _matmul_body,
        grid=(n // tile,),
        in_specs=[_row_spec(tile, din), _full_spec((din, dout)), _full_spec((1, dout))],
        out_specs=_row_spec(tile, dout),
        out_shape=jax.ShapeDtypeStruct((n, dout), jnp.float32),
        interpret=_INTERPRET,
    )(x, w, b)


def _qkv_body(x_ref, wq_ref, bq_ref, wkv_ref, bkv_ref, q_ref, kv_ref):
    xv = x_ref[...]
    q_ref[...] = _dot(xv, wq_ref[...]) + bq_ref[...]
    kv_ref[...] = _dot(xv, wkv_ref[...]) + bkv_ref[...]


def _qkv(node, wq, bq, wkv, bkv, tile):
    n = node.shape[0]
    return pl.pallas_call(
        _qkv_body,
        grid=(n // tile,),
        in_specs=[_row_spec(tile, EMB), _full_spec((EMB, EMB)), _full_spec((1, EMB)),
                  _full_spec((EMB, 2 * EMB)), _full_spec((1, 2 * EMB))],
        out_specs=[_row_spec(tile, EMB), _row_spec(tile, 2 * EMB)],
        out_shape=[jax.ShapeDtypeStruct((n, EMB), jnp.float32),
                   jax.ShapeDtypeStruct((n, 2 * EMB), jnp.float32)],
        interpret=_INTERPRET,
    )(node, wq, bq, wkv, bkv)


def _ek_em_body(ef_ref, wek_ref, bek_ref, wem_ref, bem_ref, ek_ref, em_ref):
    ef = ef_ref[...]
    ek_ref[...] = _dot(ef, wek_ref[...]) + bek_ref[...]
    em_ref[...] = _dot(ef, wem_ref[...]) + bem_ref[...]


def _ek_em(ef, wek, bek, wem, bem, tile):
    e = ef.shape[0]
    return pl.pallas_call(
        _ek_em_body,
        grid=(e // tile,),
        in_specs=[_row_spec(tile, EMB)] + [_full_spec((EMB, EMB)), _full_spec((1, EMB))] * 2,
        out_specs=[_row_spec(tile, EMB)] * 2,
        out_shape=[jax.ShapeDtypeStruct((e, EMB), jnp.float32)] * 2,
        interpret=_INTERPRET,
    )(ef, wek, bek, wem, bem)


def _conv_msg_body(qd_ref, kvs_ref, ef_ref,
                   wek_ref, bek_ref, wem_ref, bem_ref,
                   w1k_ref, w2k_ref, b2k_ref,
                   w1m_ref, w2m_ref, b2m_ref, wmsg_ref, bmsg_ref,
                   attg_ref, attb_ref, msgg_ref, msgb_ref, o_ref):
    qd = qd_ref[...]
    kvs = kvs_ref[...]
    ef = ef_ref[...]
    ks = kvs[:, :EMB]
    vs = kvs[:, EMB:]
    ek = _dot(ef, wek_ref[...]) + bek_ref[...]
    em = _dot(ef, wem_ref[...]) + bem_ref[...]
    hk = _dot(jnp.concatenate([ks, qd], axis=-1), w1k_ref[...]) + ek
    key = _dot(jax.nn.silu(hk), w2k_ref[...]) + b2k_ref[...]
    alpha = qd * key * (1.0 / jnp.sqrt(jnp.float32(EMB)))
    hm = _dot(jnp.concatenate([vs, qd], axis=-1), w1m_ref[...]) + em
    msg = _dot(jax.nn.silu(hm), w2m_ref[...]) + b2m_ref[...]
    msg = msg * jax.nn.sigmoid(_ln(alpha, attg_ref[...], attb_ref[...]))
    msg = _ln(_dot(msg, wmsg_ref[...]) + bmsg_ref[...], msgg_ref[...], msgb_ref[...])
    o_ref[...] = msg


def _conv_msg(qd, kvs, ef, wts, tile, ef_off_blocks=0):
    e = qd.shape[0]
    specs = [_row_spec(tile, EMB), _row_spec(tile, 2 * EMB),
             pl.BlockSpec((tile, EMB), lambda i: (i + ef_off_blocks, 0))]
    specs += [_full_spec(w.shape) for w in wts]
    return pl.pallas_call(
        _conv_msg_body,
        grid=(e // tile,),
        in_specs=specs,
        out_specs=_row_spec(tile, EMB),
        out_shape=jax.ShapeDtypeStruct((e, EMB), jnp.float32),
        interpret=_INTERPRET,
    )(qd, kvs, ef, *wts)


def _node_upd_body(node_ref, p0_ref, p1_ref, wcc_ref, bcc_ref, g_ref, b_ref, o_ref):
    agg = p0_ref[...] + p1_ref[...]
    out = (_dot(agg, wcc_ref[...]) + bcc_ref[...]) * g_ref[...] + b_ref[...]
    o_ref[...] = jax.nn.softplus(node_ref[...] + out)


def _node_update(node, p0, p1, wcc, bcc, g, b, tile):
    n = node.shape[0]
    return pl.pallas_call(
        _node_upd_body,
        grid=(n // tile,),
        in_specs=[_row_spec(tile, EMB)] * 3 + [_full_spec((EMB, EMB))] + [_full_spec((1, EMB))] * 3,
        out_specs=_row_spec(tile, EMB),
        out_shape=jax.ShapeDtypeStruct((n, EMB), jnp.float32),
        interpret=_INTERPRET,
    )(node, p0, p1, wcc, bcc, g, b)


def _edge_upd_body(ef_ref, l0_ref, l1_ref, l2_ref, a0_ref, a1_ref, a2_ref,
                   wq_ref, bq_ref, wk_ref, bk_ref, wv_ref, bv_ref,
                   wlen_ref, lembb_ref,
                   wke0_ref, bke0_ref, wke1_ref, bke1_ref, wke2_ref, bke2_ref,
                   wve0_ref, bve0_ref, wve1_ref, bve1_ref, wve2_ref, bve2_ref,
                   we_ref, be_ref, w1k_ref, b1k_ref, w2k_ref, b2k_ref,
                   w1m_ref, b1m_ref, w2m_ref, b2m_ref, wcc_ref, bcc_ref,
                   attg_ref, attb_ref, bng_ref, bnb_ref, o_ref):
    ed = ef_ref[...]
    w1k = w1k_ref[...]
    w1m = w1m_ref[...]
    qx = _dot(ed, wq_ref[...]) + bq_ref[...]
    kx1 = _dot(_dot(ed, wk_ref[...]) + bk_ref[...], w1k[:EMB])
    vx1 = _dot(_dot(ed, wv_ref[...]) + bv_ref[...], w1m[:EMB])
    rsq = 1.0 / jnp.sqrt(jnp.float32(EMB))
    lrefs = (l0_ref, l1_ref, l2_ref)
    arefs = (a0_ref, a1_ref, a2_ref)
    kerefs = ((wke0_ref, bke0_ref), (wke1_ref, bke1_ref), (wke2_ref, bke2_ref))
    verefs = ((wve0_ref, bve0_ref), (wve1_ref, bve1_ref), (wve2_ref, bve2_ref))
    acc = jnp.zeros_like(ed)
    for s in range(3):
        nl = jax.nn.silu(_dot(lrefs[s][...], wlen_ref[...]) + lembb_ref[s:s + 1, :])
        ky = _dot(nl, kerefs[s][0][...]) + kerefs[s][1][...]
        vy = _dot(nl, verefs[s][0][...]) + verefs[s][1][...]
        exy = _dot(arefs[s][...], we_ref[...]) + be_ref[...]
        hk = kx1 + _dot(ky, w1k[EMB:2 * EMB]) + _dot(exy, w1k[2 * EMB:]) + b1k_ref[...]
        kk = _dot(jax.nn.silu(hk), w2k_ref[...]) + b2k_ref[...]
        alpha = qx * kk * rsq
        hm = vx1 + _dot(vy, w1m[EMB:2 * EMB]) + _dot(exy, w1m[2 * EMB:]) + b1m_ref[...]
        oo = _dot(jax.nn.silu(hm), w2m_ref[...]) + b2m_ref[...]
        acc += oo * jax.nn.sigmoid(_ln(alpha, attg_ref[...], attb_ref[...]))
    res = _dot(acc, wcc_ref[...]) * (1.0 / 3.0) + bcc_ref[...]
    o_ref[...] = jax.nn.softplus(ed + res * bng_ref[...] + bnb_ref[...])


def _edge_update(ef, lens, angs, wts, tile):
    e = ef.shape[0]
    nt = e // tile
    specs = [_row_spec(tile, EMB)] * 7
    specs += [_full_spec(w.shape) for w in wts]
    return pl.pallas_call(
        _edge_upd_body,
        grid=(nt,),
        in_specs=specs,
        out_specs=_row_spec(tile, EMB),
        out_shape=jax.ShapeDtypeStruct((e, EMB), jnp.float32),
        interpret=_INTERPRET,
    )(ef, lens[0], lens[1], lens[2], angs[0], angs[1], angs[2], *wts)


def _pool_body(node_ref, b2_ref, ps_ref, cnt_ref, *, nb):
    i = pl.program_id(0)

    @pl.when(i == 0)
    def _():
        ps_ref[...] = jnp.zeros_like(ps_ref)
        cnt_ref[...] = jnp.zeros_like(cnt_ref)

    bv = b2_ref[...]
    t = bv.shape[0]
    oh = (bv == lax.broadcasted_iota(jnp.int32, (t, nb), 1)).astype(jnp.float32)
    dn = (((0,), (0,)), ((), ()))
    ps_ref[...] += lax.dot_general(oh, node_ref[...], dn,
                                   preferred_element_type=jnp.float32)
    cnt_ref[...] += lax.dot_general(oh, jnp.ones((t, EMB), jnp.float32), dn,
                                    preferred_element_type=jnp.float32)


def _pool(node, batch2d, nb, tile):
    n = node.shape[0]
    return pl.pallas_call(
        functools.partial(_pool_body, nb=nb),
        grid=(n // tile,),
        in_specs=[_row_spec(tile, EMB), pl.BlockSpec((tile, 1), lambda i: (i, 0))],
        out_specs=[_full_spec((nb, EMB)), _full_spec((nb, EMB))],
        out_shape=[jax.ShapeDtypeStruct((nb, EMB), jnp.float32),
                   jax.ShapeDtypeStruct((nb, EMB), jnp.float32)],
        interpret=_INTERPRET,
    )(node, batch2d)


def _polar_cols(c):
    for _ in range(12):
        cof = [c[4] * c[8] - c[5] * c[7], c[5] * c[6] - c[3] * c[8], c[3] * c[7] - c[4] * c[6],
               c[2] * c[7] - c[1] * c[8], c[0] * c[8] - c[2] * c[6], c[1] * c[6] - c[0] * c[7],
               c[1] * c[5] - c[2] * c[4], c[2] * c[3] - c[0] * c[5], c[0] * c[4] - c[1] * c[3]]
        det = c[0] * cof[0] + c[1] * cof[1] + c[2] * cof[2]
        adet = jnp.maximum(jnp.abs(det), 1e-30)
        mu = jnp.exp(jnp.log(adet) * (-1.0 / 3.0))
        inv_md = 1.0 / (mu * det)
        c = [0.5 * (mu * c[k] + cof[k] * inv_md) for k in range(9)]
    return c


def _head_body(ps_ref, cnt_ref, at_ref, w1_ref, b1_ref, w2_ref, b2_ref, o_ref):
    pooled = ps_ref[...] / jnp.maximum(cnt_ref[...], 1.0)
    h0 = _dot(pooled, w1_ref[...]) + b1_ref[...]
    h = jnp.where(h0 > 0, h0, jnp.exp(jnp.minimum(h0, 0.0)) - 1.0)
    o = _dot(h, w2_ref[...]) + b2_ref[...]
    a = at_ref[...]
    r = _polar_cols([a[:, k:k + 1] for k in range(9)])
    m = [o[:, k:k + 1] for k in range(9)]
    p = [r[3 * i + 0] * m[3 * 0 + j] + r[3 * i + 1] * m[3 * 1 + j] + r[3 * i + 2] * m[3 * 2 + j]
         for i in range(3) for j in range(3)]
    q = [p[3 * i + 0] * r[3 * j + 0] + p[3 * i + 1] * r[3 * j + 1] + p[3 * i + 2] * r[3 * j + 2]
         for i in range(3) for j in range(3)]
    o_ref[...] = jnp.concatenate(q + [jnp.zeros_like(q[0])] * 7, axis=-1)


def _head(ps, cnt, at16, w1, b1, w2, b2):
    nb = ps.shape[0]
    return pl.pallas_call(
        _head_body,
        grid=(1,),
        in_specs=[_full_spec((nb, EMB)), _full_spec((nb, EMB)), _full_spec((nb, 16)),
                  _full_spec((EMB, EMB)), _full_spec((1, EMB)),
                  _full_spec((EMB, 16)), _full_spec((1, 16))],
        out_specs=_full_spec((nb, 16)),
        out_shape=jax.ShapeDtypeStruct((nb, 16), jnp.float32),
        interpret=_INTERPRET,
    )(ps, cnt, at16, w1, b1, w2, b2)


def _sc_gather_pair(q_tab, kv_tab, idx_dst3, idx_src3):
    nchunk = idx_dst3.shape[1]
    per_w = nchunk * CHUNK
    e = N_WORKERS * per_w
    mesh = plsc.VectorSubcoreMesh(core_axis_name="c", subcore_axis_name="s")

    kd = 5
    ngroups = nchunk // kd
    tail = nchunk - ngroups * kd

    @functools.partial(
        pl.kernel, mesh=mesh,
        out_type=[jax.ShapeDtypeStruct((e, EMB), jnp.float32),
                  jax.ShapeDtypeStruct((e, 2 * EMB), jnp.float32)],
        scratch_types=[pltpu.VMEM((nchunk, CHUNK), jnp.int32),
                       pltpu.VMEM((nchunk, CHUNK), jnp.int32),
                       pltpu.VMEM((kd, CHUNK, EMB), jnp.float32),
                       pltpu.VMEM((kd, CHUNK, 2 * EMB), jnp.float32),
                       pltpu.SemaphoreType.DMA,
                       pltpu.SemaphoreType.DMA,
                       pltpu.SemaphoreType.DMA,
                       pltpu.SemaphoreType.DMA],
    )
    def k(qt, kvt, idxd, idxs, qd_out, kvs_out, idxd_v, idxs_v, qbuf, kvbuf,
          sgq, sgk, soq, sok):
        wid = lax.axis_index("s") * 2 + lax.axis_index("c")
        base = wid * per_w
        pltpu.sync_copy(idxd.at[wid], idxd_v)
        pltpu.sync_copy(idxs.at[wid], idxs_v)

        def group(j0, count):
            hq = [pltpu.async_copy(qt.at[idxd_v.at[j0 + b]], qbuf.at[b], sgq)
                  for b in range(count)]
            hk = [pltpu.async_copy(kvt.at[idxs_v.at[j0 + b]], kvbuf.at[b], sgk)
                  for b in range(count)]
            for h in hq:
                h.wait()
            oq = [pltpu.async_copy(qbuf.at[b],
                                   qd_out.at[pl.ds(base + (j0 + b) * CHUNK, CHUNK)], soq)
                  for b in range(count)]
            for h in hk:
                h.wait()
            ok = [pltpu.async_copy(kvbuf.at[b],
                                   kvs_out.at[pl.ds(base + (j0 + b) * CHUNK, CHUNK)], sok)
                  for b in range(count)]
            for h in oq + ok:
                h.wait()

        def body(g, carry):
            group(g * kd, kd)
            return carry

        lax.fori_loop(0, ngroups, body, 0)
        if tail:
            group(ngroups * kd, tail)

    return k(q_tab, kv_tab, idx_dst3, idx_src3)


def _sc_scatter_add(msg_a, msg_b, idx_a3, idx_b3, zeros_tab):
    nca = idx_a3.shape[1]
    ncb = idx_b3.shape[1]
    np_ = zeros_tab.shape[0]
    rows_per_sub = np_ // 16
    mesh = plsc.VectorSubcoreMesh(core_axis_name="c", subcore_axis_name="s")

    kd = 5

    @functools.partial(
        pl.kernel, mesh=mesh,
        out_type=jax.ShapeDtypeStruct((2, np_, EMB), jnp.float32),
        scratch_types=[pltpu.VMEM_SHARED((np_, EMB), jnp.float32),
                       pltpu.VMEM((nca, CHUNK), jnp.int32),
                       pltpu.VMEM((ncb, CHUNK), jnp.int32),
                       pltpu.VMEM((kd * CHUNK, EMB), jnp.float32),
                       pltpu.SemaphoreType.DMA],
    )
    def k(msg_ha, msg_hb, idxa, idxb, zeros_h, out, shared, idxa_v, idxb_v, mbuf, ssa):
        cid = lax.axis_index("c")
        sid = lax.axis_index("s")
        wid = sid * 2 + cid
        row0 = sid * rows_per_sub
        pltpu.sync_copy(zeros_h.at[pl.ds(row0, rows_per_sub)],
                        shared.at[pl.ds(row0, rows_per_sub)])
        plsc.subcore_barrier()
        pltpu.sync_copy(idxa.at[wid], idxa_v)
        pltpu.sync_copy(idxb.at[wid], idxb_v)

        def run(msg_h, idx_v, nchunk):
            base = wid * nchunk * CHUNK
            ngroups = nchunk // kd
            tail = nchunk - ngroups * kd

            def group(j0, count):
                pltpu.sync_copy(msg_h.at[pl.ds(base + j0 * CHUNK, count * CHUNK)],
                                mbuf.at[pl.ds(0, count * CHUNK)])
                hs = [pltpu.async_copy(mbuf.at[pl.ds(b * CHUNK, CHUNK)],
                                       shared.at[idx_v.at[j0 + b]], ssa, add=True)
                      for b in range(count)]
                for h in hs:
                    h.wait()

            def body(g, carry):
                group(g * kd, kd)
                return carry

            lax.fori_loop(0, ngroups, body, 0)
            if tail:
                group(ngroups * kd, tail)

        run(msg_ha, idxa_v, nca)
        run(msg_hb, idxb_v, ncb)
        plsc.subcore_barrier()
        pltpu.sync_copy(shared.at[pl.ds(row0, rows_per_sub)],
                        out.at[cid, pl.ds(row0, rows_per_sub)])

    return k(msg_a, msg_b, idx_a3, idx_b3, zeros_tab)


def kernel(x, edge_attr, edge_nei, pos, cell, edge_index, batch, params):
    p = params
    n = x.shape[0]
    e = edge_attr.shape[0]
    nb = cell.shape[0]
    np_ = -(-n // 2048) * 2048
    t_e = 1600 if e % 1600 == 0 else 400
    t_r = 1280 if e % 1280 == 0 else 128
    t_n = 512
    f32 = jnp.float32

    xp = jnp.zeros((np_, EMB), f32).at[:n, :x.shape[1]].set(x)
    batch_p = jnp.concatenate([batch.astype(jnp.int32),
                               jnp.full((np_ - n,), nb, jnp.int32)])
    batch2d = batch_p.reshape(np_, 1)
    src = edge_index[0].astype(jnp.int32)
    dst = edge_index[1].astype(jnp.int32)
    nei9 = edge_nei.reshape(e, 9)

    def b1(arr):
        return arr.reshape(1, -1)

    w_atom = jnp.zeros((EMB, EMB), f32).at[:p['atom_emb']['w'].shape[0]].set(p['atom_emb']['w'])
    node = _node_matmul(xp, w_atom, b1(p['atom_emb']['b']), t_n)

    t_h = 1280
    e_a = (e // 2 // t_h) * t_h
    e_b = e - e_a
    if _USE_SC:
        idx_dst_a = dst[:e_a].reshape(N_WORKERS, -1, CHUNK)
        idx_src_a = src[:e_a].reshape(N_WORKERS, -1, CHUNK)
        idx_dst_b = dst[e_a:].reshape(N_WORKERS, -1, CHUNK)
        idx_src_b = src[e_a:].reshape(N_WORKERS, -1, CHUNK)
        zeros_tab = jnp.zeros((np_, EMB), f32)

    def gather_stage(node, cp):
        wkv = jnp.concatenate([cp['k']['w'], cp['v']['w']], axis=1)
        bkv = jnp.concatenate([cp['k']['b'], cp['v']['b']]).reshape(1, -1)
        q_tab, kv_tab = _qkv(node, cp['q']['w'], b1(cp['q']['b']), wkv, bkv, t_n)
        if _USE_SC:
            ga = _sc_gather_pair(q_tab, kv_tab, idx_dst_a, idx_src_a)
            gb = _sc_gather_pair(q_tab, kv_tab, idx_dst_b, idx_src_b)
            return ga[0], ga[1], gb[0], gb[1]
        return (q_tab[dst[:e_a]], kv_tab[src[:e_a]],
                q_tab[dst[e_a:]], kv_tab[src[e_a:]])

    def msg_stage(g4, ef, cp):
        qd_a, kvs_a, qd_b, kvs_b = g4
        w1k = cp['ku']['l1']['w']
        w1m = cp['mu']['l1']['w']
        wek = cp['e']['w'] @ w1k[2 * EMB:]
        bek = (cp['e']['b'] @ w1k[2 * EMB:] + cp['ku']['l1']['b']).reshape(1, -1)
        wem = cp['e']['w'] @ w1m[2 * EMB:]
        bem = (cp['e']['b'] @ w1m[2 * EMB:] + cp['mu']['l1']['b']).reshape(1, -1)
        wts = [wek, bek, wem, bem,
               w1k[:2 * EMB],
               cp['ku']['l2']['w'], b1(cp['ku']['l2']['b']),
               w1m[:2 * EMB],
               cp['mu']['l2']['w'], b1(cp['mu']['l2']['b']),
               cp['msg']['w'], b1(cp['msg']['b']),
               b1(cp['att_ln_g']), b1(cp['att_ln_b']),
               b1(cp['msg_ln_g']), b1(cp['msg_ln_b'])]
        msg_a = _conv_msg(qd_a, kvs_a, ef, wts, t_h)
        msg_b = _conv_msg(qd_b, kvs_b, ef, wts, t_h, ef_off_blocks=e_a // t_h)
        if _USE_SC:
            part = _sc_scatter_add(msg_a, msg_b, idx_dst_a, idx_dst_b, zeros_tab)
            return part[0], part[1]
        p = (jax.ops.segment_sum(msg_a, dst[:e_a], num_segments=np_)
             + jax.ops.segment_sum(msg_b, dst[e_a:], num_segments=np_))
        return p, None

    def upd_stage(node, p0, p1, cp):
        if p1 is None:
            p1 = jnp.zeros_like(p0)
        return _node_update(node, p0, p1, cp['cc']['w'], b1(cp['cc']['b']),
                            b1(cp['bn_g']), b1(cp['bn_b']), t_n)

    def conv(node, ef, cp):
        g4 = gather_stage(node, cp)
        p0, p1 = msg_stage(g4, ef, cp)
        return upd_stage(node, p0, p1, cp)

    ea_t = edge_attr.T
    nei_t = nei9.T
    cp0 = p['att0']
    g40 = gather_stage(node, cp0)
    ef = _rbf_edge(ea_t, p['rbf_lin']['w'], b1(p['rbf_lin']['b']), t_r)
    lens = _rbf_len(nei_t, p['rbf_lin']['w'], b1(p['rbf_lin']['b']), t_r)
    pt0, pt1 = msg_stage(g40, ef, cp0)
    angs = _rbf_ang(nei_t, ea_t, p['rbf_angle_lin']['w'],
                    b1(p['rbf_angle_lin']['b']), t_r)
    if pt1 is not None:
        bar = jax.lax.optimization_barrier((pt0, pt1) + tuple(angs))
        pt0, pt1 = bar[0], bar[1]
        angs = bar[2:5]

    ep = p['edge_upd']
    wlen_a = ep['len']['w'][:EMB]
    lembb = ep['lemb'] @ ep['len']['w'][EMB:] + ep['len']['b']
    lembb8 = jnp.zeros((8, EMB), f32).at[:3].set(lembb)
    ewts = [ep['q']['w'], b1(ep['q']['b']), ep['k']['w'], b1(ep['k']['b']),
            ep['v']['w'], b1(ep['v']['b']), wlen_a, lembb8,
            ep['ke1']['w'], b1(ep['ke1']['b']), ep['ke2']['w'], b1(ep['ke2']['b']),
            ep['ke3']['w'], b1(ep['ke3']['b']),
            ep['ve1']['w'], b1(ep['ve1']['b']), ep['ve2']['w'], b1(ep['ve2']['b']),
            ep['ve3']['w'], b1(ep['ve3']['b']),
            ep['e']['w'], b1(ep['e']['b']),
            ep['ku']['l1']['w'], b1(ep['ku']['l1']['b']),
            ep['ku']['l2']['w'], b1(ep['ku']['l2']['b']),
            ep['mu']['l1']['w'], b1(ep['mu']['l1']['b']),
            ep['mu']['l2']['w'], b1(ep['mu']['l2']['b']),
            ep['cc']['w'], b1(ep['cc']['b']),
            b1(ep['att_ln_g']), b1(ep['att_ln_b']),
            b1(ep['bn_g']), b1(ep['bn_b'])]
    node = upd_stage(node, pt0, pt1, cp0)
    cp1 = p['att1']
    g41 = gather_stage(node, cp1)
    ef = _edge_update(ef, lens, angs, ewts, t_e)
    pt0, pt1 = msg_stage(g41, ef, cp1)
    node = upd_stage(node, pt0, pt1, cp1)
    node = conv(node, ef, p['att2'])
    node = conv(node, ef, p['att3'])

    ps, cnt = _pool(node, batch2d, nb, t_n)
    at9 = jnp.swapaxes(cell, -2, -1).reshape(nb, 9)
    at16 = jnp.zeros((nb, 16), f32).at[:, :9].set(at9)
    w2 = jnp.zeros((EMB, 16), f32).at[:, :9].set(p['fc2']['w'])
    b2 = jnp.zeros((1, 16), f32).at[0, :9].set(p['fc2']['b'])
    outf = _head(ps, cnt, at16, p['fc1']['w'], b1(p['fc1']['b']), w2, b2)
    return outf[:, :9].reshape(nb, 3, 3)


# speedup vs baseline: 1.0439x; 1.0221x over previous
import functools

import jax
import jax.numpy as jnp
from jax import lax
from jax.experimental import pallas as pl
from jax.experimental.pallas import tpu as pltpu
from jax.experimental.pallas import tpu_sc as plsc

_INTERPRET = False
_USE_SC = True

EMB = 128
BINS = 512
NP_NODES = 10240
N_WORKERS = 32
CHUNK = 40


def _ln(xv, g, b, eps=1e-5):
    mu = jnp.mean(xv, axis=-1, keepdims=True)
    xc = xv - mu
    var = jnp.mean(xc * xc, axis=-1, keepdims=True)
    return xc * lax.rsqrt(var + eps) * g + b


def _dot(a, b):
    return jnp.dot(a, b, preferred_element_type=jnp.float32)


def _expansion_t(f, vmin, delta):
    g = 1.0 / delta
    c = vmin + delta * lax.broadcasted_iota(jnp.int32, (BINS, 1), 0).astype(jnp.float32)
    d = f - c
    return jnp.exp(d * (d * (-g)))


def _dot_t(z, w):
    return jax.lax.dot_general(z, w, (((0,), (0,)), ((), ())),
                               preferred_element_type=jnp.float32)


def _row_spec(t, w):
    return pl.BlockSpec((t, w), lambda i: (i, 0))


def _full_spec(shape):
    nd = len(shape)
    return pl.BlockSpec(shape, lambda i: (0,) * nd)


def _rbf_edge_body(ea_ref, w_ref, b_ref, o_ref, *, vmin, vmax):
    a = ea_ref[...]
    n2 = jnp.sum(a * a, axis=0, keepdims=True)
    f = -0.75 / jnp.sqrt(n2)
    delta = (vmax - vmin) / (BINS - 1)
    z = _expansion_t(f, vmin, delta)
    o_ref[...] = jax.nn.softplus(_dot_t(z, w_ref[...]) + b_ref[...])


def _rbf_edge(ea_t, w, b, tile):
    e = ea_t.shape[1]
    return pl.pallas_call(
        functools.partial(_rbf_edge_body, vmin=-4.0, vmax=0.0),
        grid=(e // tile,),
        in_specs=[pl.BlockSpec((3, tile), lambda i: (0, i)),
                  _full_spec((BINS, EMB)), _full_spec((1, EMB))],
        out_specs=_row_spec(tile, EMB),
        out_shape=jax.ShapeDtypeStruct((e, EMB), jnp.float32),
        interpret=_INTERPRET,
    )(ea_t, w, b)


def _rbf_len_body(nei_ref, wl_ref, bl_ref, l0_ref, l1_ref, l2_ref):
    nei = nei_ref[...]
    wl = wl_ref[...]
    louts = (l0_ref, l1_ref, l2_ref)
    dl = 4.0 / (BINS - 1)
    for s in range(3):
        r1 = nei[3 * s:3 * s + 3, :]
        n1 = jnp.sqrt(jnp.sum(r1 * r1, axis=0, keepdims=True))
        nlen = -0.75 / n1
        zl = _expansion_t(nlen, -4.0, dl)
        louts[s][...] = jax.nn.softplus(_dot_t(zl, wl) + bl_ref[...])


def _rbf_len(nei_t, wl, bl, tile):
    e = nei_t.shape[1]
    return pl.pallas_call(
        _rbf_len_body,
        grid=(e // tile,),
        in_specs=[pl.BlockSpec((9, tile), lambda i: (0, i)),
                  _full_spec((BINS, EMB)), _full_spec((1, EMB))],
        out_specs=[_row_spec(tile, EMB)] * 3,
        out_shape=[jax.ShapeDtypeStruct((e, EMB), jnp.float32)] * 3,
        interpret=_INTERPRET,
    )(nei_t, wl, bl)


def _rbf_ang_body(nei_ref, ea_ref, wa_ref, ba_ref, a0_ref, a1_ref, a2_ref):
    nei = nei_ref[...]
    r2 = ea_ref[...]
    n2 = jnp.sqrt(jnp.sum(r2 * r2, axis=0, keepdims=True))
    wa = wa_ref[...]
    aouts = (a0_ref, a1_ref, a2_ref)
    da = 2.0 / (BINS - 1)
    for s in range(3):
        r1 = nei[3 * s:3 * s + 3, :]
        dot = jnp.sum(r1 * r2, axis=0, keepdims=True)
        n1 = jnp.sqrt(jnp.sum(r1 * r1, axis=0, keepdims=True))
        nprod = n1 * n2
        cos = jnp.clip(dot / (nprod + 1e-8), -1.0, 1.0)
        cos = jnp.where(nprod == 0, 1.0, cos)
        za = _expansion_t(cos, -1.0, da)
        aouts[s][...] = jax.nn.softplus(_dot_t(za, wa) + ba_ref[...])


def _rbf_ang(nei_t, ea_t, wa, ba, tile):
    e = nei_t.shape[1]
    return pl.pallas_call(
        _rbf_ang_body,
        grid=(e // tile,),
        in_specs=[pl.BlockSpec((9, tile), lambda i: (0, i)),
                  pl.BlockSpec((3, tile), lambda i: (0, i)),
                  _full_spec((BINS, EMB)), _full_spec((1, EMB))],
        out_specs=[_row_spec(tile, EMB)] * 3,
        out_shape=[jax.ShapeDtypeStruct((e, EMB), jnp.float32)] * 3,
        interpret=_INTERPRET,
    )(nei_t, ea_t, wa, ba)


def _matmul_body(x_ref, w_ref, b_ref, o_ref):
    o_ref[...] = _dot(x_ref[...], w_ref[...]) + b_ref[...]


def _node_matmul(x, w, b, tile):
    n = x.shape[0]
    din, dout = w.shape
    return pl.pallas_call(
        _matmul_body,
        grid=(n // tile,),
        in_specs=[_row_spec(tile, din), _full_spec((din, dout)), _full_spec((1, dout))],
        out_specs=_row_spec(tile, dout),
        out_shape=jax.ShapeDtypeStruct((n, dout), jnp.float32),
        interpret=_INTERPRET,
    )(x, w, b)


def _qkv_body(x_ref, wq_ref, bq_ref, wkv_ref, bkv_ref, q_ref, kv_ref):
    xv = x_ref[...]
    q_ref[...] = _dot(xv, wq_ref[...]) + bq_ref[...]
    kv_ref[...] = _dot(xv, wkv_ref[...]) + bkv_ref[...]


def _qkv(node, wq, bq, wkv, bkv, tile):
    n = node.shape[0]
    return pl.pallas_call(
        _qkv_body,
        grid=(n // tile,),
        in_specs=[_row_spec(tile, EMB), _full_spec((EMB, EMB)), _full_spec((1, EMB)),
                  _full_spec((EMB, 2 * EMB)), _full_spec((1, 2 * EMB))],
        out_specs=[_row_spec(tile, EMB), _row_spec(tile, 2 * EMB)],
        out_shape=[jax.ShapeDtypeStruct((n, EMB), jnp.float32),
                   jax.ShapeDtypeStruct((n, 2 * EMB), jnp.float32)],
        interpret=_INTERPRET,
    )(node, wq, bq, wkv, bkv)


def _ek_em_body(ef_ref, wek_ref, bek_ref, wem_ref, bem_ref, ek_ref, em_ref):
    ef = ef_ref[...]
    ek_ref[...] = _dot(ef, wek_ref[...]) + bek_ref[...]
    em_ref[...] = _dot(ef, wem_ref[...]) + bem_ref[...]


def _ek_em(ef, wek, bek, wem, bem, tile):
    e = ef.shape[0]
    return pl.pallas_call(
        _ek_em_body,
        grid=(e // tile,),
        in_specs=[_row_spec(tile, EMB)] + [_full_spec((EMB, EMB)), _full_spec((1, EMB))] * 2,
        out_specs=[_row_spec(tile, EMB)] * 2,
        out_shape=[jax.ShapeDtypeStruct((e, EMB), jnp.float32)] * 2,
        interpret=_INTERPRET,
    )(ef, wek, bek, wem, bem)


def _conv_msg_body(qd_ref, kvs_ref, ef_ref,
                   wek_ref, bek_ref, wem_ref, bem_ref,
                   w1k_ref, w2k_ref, b2k_ref,
                   w1m_ref, w2m_ref, b2m_ref, wmsg_ref, bmsg_ref,
                   attg_ref, attb_ref, msgg_ref, msgb_ref, o_ref):
    qd = qd_ref[...]
    kvs = kvs_ref[...]
    ef = ef_ref[...]
    ks = kvs[:, :EMB]
    vs = kvs[:, EMB:]
    ek = _dot(ef, wek_ref[...]) + bek_ref[...]
    em = _dot(ef, wem_ref[...]) + bem_ref[...]
    hk = _dot(jnp.concatenate([ks, qd], axis=-1), w1k_ref[...]) + ek
    key = _dot(jax.nn.silu(hk), w2k_ref[...]) + b2k_ref[...]
    alpha = qd * key * (1.0 / jnp.sqrt(jnp.float32(EMB)))
    hm = _dot(jnp.concatenate([vs, qd], axis=-1), w1m_ref[...]) + em
    msg = _dot(jax.nn.silu(hm), w2m_ref[...]) + b2m_ref[...]
    msg = msg * jax.nn.sigmoid(_ln(alpha, attg_ref[...], attb_ref[...]))
    msg = _ln(_dot(msg, wmsg_ref[...]) + bmsg_ref[...], msgg_ref[...], msgb_ref[...])
    o_ref[...] = msg


def _conv_msg(qd, kvs, ef, wts, tile, ef_off_blocks=0):
    e = qd.shape[0]
    specs = [_row_spec(tile, EMB), _row_spec(tile, 2 * EMB),
             pl.BlockSpec((tile, EMB), lambda i: (i + ef_off_blocks, 0))]
    specs += [_full_spec(w.shape) for w in wts]
    return pl.pallas_call(
        _conv_msg_body,
        grid=(e // tile,),
        in_specs=specs,
        out_specs=_row_spec(tile, EMB),
        out_shape=jax.ShapeDtypeStruct((e, EMB), jnp.float32),
        interpret=_INTERPRET,
    )(qd, kvs, ef, *wts)


def _node_upd_body(node_ref, p0_ref, p1_ref, wcc_ref, bcc_ref, g_ref, b_ref, o_ref):
    agg = p0_ref[...] + p1_ref[...]
    out = (_dot(agg, wcc_ref[...]) + bcc_ref[...]) * g_ref[...] + b_ref[...]
    o_ref[...] = jax.nn.softplus(node_ref[...] + out)


def _node_update(node, p0, p1, wcc, bcc, g, b, tile):
    n = node.shape[0]
    return pl.pallas_call(
        _node_upd_body,
        grid=(n // tile,),
        in_specs=[_row_spec(tile, EMB)] * 3 + [_full_spec((EMB, EMB))] + [_full_spec((1, EMB))] * 3,
        out_specs=_row_spec(tile, EMB),
        out_shape=jax.ShapeDtypeStruct((n, EMB), jnp.float32),
        interpret=_INTERPRET,
    )(node, p0, p1, wcc, bcc, g, b)


def _edge_upd_body(ef_ref, l0_ref, l1_ref, l2_ref, a0_ref, a1_ref, a2_ref,
                   wq_ref, bq_ref, wk_ref, bk_ref, wv_ref, bv_ref,
                   wlen_ref, lembb_ref,
                   wke0_ref, bke0_ref, wke1_ref, bke1_ref, wke2_ref, bke2_ref,
                   wve0_ref, bve0_ref, wve1_ref, bve1_ref, wve2_ref, bve2_ref,
                   we_ref, be_ref, w1k_ref, b1k_ref, w2k_ref, b2k_ref,
                   w1m_ref, b1m_ref, w2m_ref, b2m_ref, wcc_ref, bcc_ref,
                   attg_ref, attb_ref, bng_ref, bnb_ref, o_ref):
    ed = ef_ref[...]
    w1k = w1k_ref[...]
    w1m = w1m_ref[...]
    qx = _dot(ed, wq_ref[...]) + bq_ref[...]
    kx1 = _dot(_dot(ed, wk_ref[...]) + bk_ref[...], w1k[:EMB])
    vx1 = _dot(_dot(ed, wv_ref[...]) + bv_ref[...], w1m[:EMB])
    rsq = 1.0 / jnp.sqrt(jnp.float32(EMB))
    lrefs = (l0_ref, l1_ref, l2_ref)
    arefs = (a0_ref, a1_ref, a2_ref)
    kerefs = ((wke0_ref, bke0_ref), (wke1_ref, bke1_ref), (wke2_ref, bke2_ref))
    verefs = ((wve0_ref, bve0_ref), (wve1_ref, bve1_ref), (wve2_ref, bve2_ref))
    acc = jnp.zeros_like(ed)
    for s in range(3):
        nl = jax.nn.silu(_dot(lrefs[s][...], wlen_ref[...]) + lembb_ref[s:s + 1, :])
        ky = _dot(nl, kerefs[s][0][...]) + kerefs[s][1][...]
        vy = _dot(nl, verefs[s][0][...]) + verefs[s][1][...]
        exy = _dot(arefs[s][...], we_ref[...]) + be_ref[...]
        hk = kx1 + _dot(ky, w1k[EMB:2 * EMB]) + _dot(exy, w1k[2 * EMB:]) + b1k_ref[...]
        kk = _dot(jax.nn.silu(hk), w2k_ref[...]) + b2k_ref[...]
        alpha = qx * kk * rsq
        hm = vx1 + _dot(vy, w1m[EMB:2 * EMB]) + _dot(exy, w1m[2 * EMB:]) + b1m_ref[...]
        oo = _dot(jax.nn.silu(hm), w2m_ref[...]) + b2m_ref[...]
        acc += oo * jax.nn.sigmoid(_ln(alpha, attg_ref[...], attb_ref[...]))
    res = _dot(acc, wcc_ref[...]) * (1.0 / 3.0) + bcc_ref[...]
    o_ref[...] = jax.nn.softplus(ed + res * bng_ref[...] + bnb_ref[...])


def _edge_update(ef, lens, angs, wts, tile):
    e = ef.shape[0]
    nt = e // tile
    specs = [_row_spec(tile, EMB)] * 7
    specs += [_full_spec(w.shape) for w in wts]
    return pl.pallas_call(
        _edge_upd_body,
        grid=(nt,),
        in_specs=specs,
        out_specs=_row_spec(tile, EMB),
        out_shape=jax.ShapeDtypeStruct((e, EMB), jnp.float32),
        interpret=_INTERPRET,
    )(ef, lens[0], lens[1], lens[2], angs[0], angs[1], angs[2], *wts)


def _pool_body(node_ref, b2_ref, ps_ref, cnt_ref, *, nb):
    i = pl.program_id(0)

    @pl.when(i == 0)
    def _():
        ps_ref[...] = jnp.zeros_like(ps_ref)
        cnt_ref[...] = jnp.zeros_like(cnt_ref)

    bv = b2_ref[...]
    t = bv.shape[0]
    oh = (bv == lax.broadcasted_iota(jnp.int32, (t, nb), 1)).astype(jnp.float32)
    dn = (((0,), (0,)), ((), ()))
    ps_ref[...] += lax.dot_general(oh, node_ref[...], dn,
                                   preferred_element_type=jnp.float32)
    cnt_ref[...] += lax.dot_general(oh, jnp.ones((t, EMB), jnp.float32), dn,
                                    preferred_element_type=jnp.float32)


def _pool(node, batch2d, nb, tile):
    n = node.shape[0]
    return pl.pallas_call(
        functools.partial(_pool_body, nb=nb),
        grid=(n // tile,),
        in_specs=[_row_spec(tile, EMB), pl.BlockSpec((tile, 1), lambda i: (i, 0))],
        out_specs=[_full_spec((nb, EMB)), _full_spec((nb, EMB))],
        out_shape=[jax.ShapeDtypeStruct((nb, EMB), jnp.float32),
                   jax.ShapeDtypeStruct((nb, EMB), jnp.float32)],
        interpret=_INTERPRET,
    )(node, batch2d)


def _polar_cols(c):
    for _ in range(12):
        cof = [c[4] * c[8] - c[5] * c[7], c[5] * c[6] - c[3] * c[8], c[3] * c[7] - c[4] * c[6],
               c[2] * c[7] - c[1] * c[8], c[0] * c[8] - c[2] * c[6], c[1] * c[6] - c[0] * c[7],
               c[1] * c[5] - c[2] * c[4], c[2] * c[3] - c[0] * c[5], c[0] * c[4] - c[1] * c[3]]
        det = c[0] * cof[0] + c[1] * cof[1] + c[2] * cof[2]
        adet = jnp.maximum(jnp.abs(det), 1e-30)
        mu = jnp.exp(jnp.log(adet) * (-1.0 / 3.0))
        inv_md = 1.0 / (mu * det)
        c = [0.5 * (mu * c[k] + cof[k] * inv_md) for k in range(9)]
    return c


def _head_body(ps_ref, cnt_ref, at_ref, w1_ref, b1_ref, w2_ref, b2_ref, o_ref):
    pooled = ps_ref[...] / jnp.maximum(cnt_ref[...], 1.0)
    h0 = _dot(pooled, w1_ref[...]) + b1_ref[...]
    h = jnp.where(h0 > 0, h0, jnp.exp(jnp.minimum(h0, 0.0)) - 1.0)
    o = _dot(h, w2_ref[...]) + b2_ref[...]
    a = at_ref[...]
    r = _polar_cols([a[:, k:k + 1] for k in range(9)])
    m = [o[:, k:k + 1] for k in range(9)]
    p = [r[3 * i + 0] * m[3 * 0 + j] + r[3 * i + 1] * m[3 * 1 + j] + r[3 * i + 2] * m[3 * 2 + j]
         for i in range(3) for j in range(3)]
    q = [p[3 * i + 0] * r[3 * j + 0] + p[3 * i + 1] * r[3 * j + 1] + p[3 * i + 2] * r[3 * j + 2]
         for i in range(3) for j in range(3)]
    o_ref[...] = jnp.concatenate(q + [jnp.zeros_like(q[0])] * 7, axis=-1)


def _head(ps, cnt, at16, w1, b1, w2, b2):
    nb = ps.shape[0]
    return pl.pallas_call(
        _head_body,
        grid=(1,),
        in_specs=[_full_spec((nb, EMB)), _full_spec((nb, EMB)), _full_spec((nb, 16)),
                  _full_spec((EMB, EMB)), _full_spec((1, EMB)),
                  _full_spec((EMB, 16)), _full_spec((1, 16))],
        out_specs=_full_spec((nb, 16)),
        out_shape=jax.ShapeDtypeStruct((nb, 16), jnp.float32),
        interpret=_INTERPRET,
    )(ps, cnt, at16, w1, b1, w2, b2)


def _sc_gather_pair(q_tab, kv_tab, idx_dst3, idx_src3):
    nchunk = idx_dst3.shape[1]
    per_w = nchunk * CHUNK
    e = N_WORKERS * per_w
    mesh = plsc.VectorSubcoreMesh(core_axis_name="c", subcore_axis_name="s")

    kd = 5
    ngroups = nchunk // kd
    tail = nchunk - ngroups * kd

    @functools.partial(
        pl.kernel, mesh=mesh,
        out_type=[jax.ShapeDtypeStruct((e, EMB), jnp.float32),
                  jax.ShapeDtypeStruct((e, 2 * EMB), jnp.float32)],
        scratch_types=[pltpu.VMEM((nchunk, CHUNK), jnp.int32),
                       pltpu.VMEM((nchunk, CHUNK), jnp.int32),
                       pltpu.VMEM((kd, CHUNK, EMB), jnp.float32),
                       pltpu.VMEM((kd, CHUNK, 2 * EMB), jnp.float32),
                       pltpu.SemaphoreType.DMA,
                       pltpu.SemaphoreType.DMA,
                       pltpu.SemaphoreType.DMA,
                       pltpu.SemaphoreType.DMA],
    )
    def k(qt, kvt, idxd, idxs, qd_out, kvs_out, idxd_v, idxs_v, qbuf, kvbuf,
          sgq, sgk, soq, sok):
        wid = lax.axis_index("s") * 2 + lax.axis_index("c")
        base = wid * per_w
        pltpu.sync_copy(idxd.at[wid], idxd_v)
        pltpu.sync_copy(idxs.at[wid], idxs_v)

        def group(j0, count):
            hq = [pltpu.async_copy(qt.at[idxd_v.at[j0 + b]], qbuf.at[b], sgq)
                  for b in range(count)]
            hk = [pltpu.async_copy(kvt.at[idxs_v.at[j0 + b]], kvbuf.at[b], sgk)
                  for b in range(count)]
            for h in hq:
                h.wait()
            oq = [pltpu.async_copy(qbuf.at[b],
                                   qd_out.at[pl.ds(base + (j0 + b) * CHUNK, CHUNK)], soq)
                  for b in range(count)]
            for h in hk:
                h.wait()
            ok = [pltpu.async_copy(kvbuf.at[b],
                                   kvs_out.at[pl.ds(base + (j0 + b) * CHUNK, CHUNK)], sok)
                  for b in range(count)]
            for h in oq + ok:
                h.wait()

        def body(g, carry):
            group(g * kd, kd)
            return carry

        lax.fori_loop(0, ngroups, body, 0)
        if tail:
            group(ngroups * kd, tail)

    return k(q_tab, kv_tab, idx_dst3, idx_src3)


def _sc_scatter_add(msg_a, msg_b, idx_a3, idx_b3, zeros_tab):
    nca = idx_a3.shape[1]
    ncb = idx_b3.shape[1]
    np_ = zeros_tab.shape[0]
    rows_per_sub = np_ // 16
    mesh = plsc.VectorSubcoreMesh(core_axis_name="c", subcore_axis_name="s")

    kd = 5

    @functools.partial(
        pl.kernel, mesh=mesh,
        out_type=jax.ShapeDtypeStruct((2, np_, EMB), jnp.float32),
        scratch_types=[pltpu.VMEM_SHARED((np_, EMB), jnp.float32),
                       pltpu.VMEM((nca, CHUNK), jnp.int32),
                       pltpu.VMEM((ncb, CHUNK), jnp.int32),
                       pltpu.VMEM((5 * CHUNK, EMB), jnp.float32),
                       pltpu.SemaphoreType.DMA],
    )
    def k(msg_ha, msg_hb, idxa, idxb, zeros_h, out, shared, idxa_v, idxb_v, mbuf, ssa):
        cid = lax.axis_index("c")
        sid = lax.axis_index("s")
        wid = sid * 2 + cid
        row0 = sid * rows_per_sub
        pltpu.sync_copy(zeros_h.at[pl.ds(row0, rows_per_sub)],
                        shared.at[pl.ds(row0, rows_per_sub)])
        plsc.subcore_barrier()
        pltpu.sync_copy(idxa.at[wid], idxa_v)
        pltpu.sync_copy(idxb.at[wid], idxb_v)

        def run(msg_h, idx_v, nchunk):
            base = wid * nchunk * CHUNK
            ngroups = nchunk // kd
            tail = nchunk - ngroups * kd

            def group(j0, count):
                pltpu.sync_copy(msg_h.at[pl.ds(base + j0 * CHUNK, count * CHUNK)],
                                mbuf.at[pl.ds(0, count * CHUNK)])
                hs = [pltpu.async_copy(mbuf.at[pl.ds(b * CHUNK, CHUNK)],
                                       shared.at[idx_v.at[j0 + b]], ssa, add=True)
                      for b in range(count)]
                for h in hs:
                    h.wait()

            def body(g, carry):
                group(g * kd, kd)
                return carry

            lax.fori_loop(0, ngroups, body, 0)
            if tail:
                group(ngroups * kd, tail)

        run(msg_ha, idxa_v, nca)
        run(msg_hb, idxb_v, ncb)
        plsc.subcore_barrier()
        pltpu.sync_copy(shared.at[pl.ds(row0, rows_per_sub)],
                        out.at[cid, pl.ds(row0, rows_per_sub)])

    return k(msg_a, msg_b, idx_a3, idx_b3, zeros_tab)


def kernel(x, edge_attr, edge_nei, pos, cell, edge_index, batch, params):
    p = params
    n = x.shape[0]
    e = edge_attr.shape[0]
    nb = cell.shape[0]
    np_ = -(-n // 2048) * 2048
    t_e = 1600 if e % 1600 == 0 else 400
    t_r = 1280 if e % 1280 == 0 else 128
    t_n = 2048 if n == 10000 else 512
    f32 = jnp.float32

    xp = jnp.zeros((np_, EMB), f32).at[:n, :x.shape[1]].set(x)
    batch_p = jnp.concatenate([batch.astype(jnp.int32),
                               jnp.full((np_ - n,), nb, jnp.int32)])
    batch2d = batch_p.reshape(np_, 1)
    src = edge_index[0].astype(jnp.int32)
    dst = edge_index[1].astype(jnp.int32)
    nei9 = edge_nei.reshape(e, 9)

    def b1(arr):
        return arr.reshape(1, -1)

    w_atom = jnp.zeros((EMB, EMB), f32).at[:p['atom_emb']['w'].shape[0]].set(p['atom_emb']['w'])
    node = _node_matmul(xp, w_atom, b1(p['atom_emb']['b']), t_n)

    t_h = 1280
    e_a = (e // 2 // t_h) * t_h
    e_b = e - e_a
    if _USE_SC:
        idx_dst_a = dst[:e_a].reshape(N_WORKERS, -1, CHUNK)
        idx_src_a = src[:e_a].reshape(N_WORKERS, -1, CHUNK)
        idx_dst_b = dst[e_a:].reshape(N_WORKERS, -1, CHUNK)
        idx_src_b = src[e_a:].reshape(N_WORKERS, -1, CHUNK)
        zeros_tab = jnp.zeros((np_, EMB), f32)

    def gather_stage(node, cp):
        wkv = jnp.concatenate([cp['k']['w'], cp['v']['w']], axis=1)
        bkv = jnp.concatenate([cp['k']['b'], cp['v']['b']]).reshape(1, -1)
        q_tab, kv_tab = _qkv(node, cp['q']['w'], b1(cp['q']['b']), wkv, bkv, t_n)
        if _USE_SC:
            ga = _sc_gather_pair(q_tab, kv_tab, idx_dst_a, idx_src_a)
            gb = _sc_gather_pair(q_tab, kv_tab, idx_dst_b, idx_src_b)
            return ga[0], ga[1], gb[0], gb[1]
        return (q_tab[dst[:e_a]], kv_tab[src[:e_a]],
                q_tab[dst[e_a:]], kv_tab[src[e_a:]])

    def msg_stage(g4, ef, cp):
        qd_a, kvs_a, qd_b, kvs_b = g4
        w1k = cp['ku']['l1']['w']
        w1m = cp['mu']['l1']['w']
        wek = cp['e']['w'] @ w1k[2 * EMB:]
        bek = (cp['e']['b'] @ w1k[2 * EMB:] + cp['ku']['l1']['b']).reshape(1, -1)
        wem = cp['e']['w'] @ w1m[2 * EMB:]
        bem = (cp['e']['b'] @ w1m[2 * EMB:] + cp['mu']['l1']['b']).reshape(1, -1)
        wts = [wek, bek, wem, bem,
               w1k[:2 * EMB],
               cp['ku']['l2']['w'], b1(cp['ku']['l2']['b']),
               w1m[:2 * EMB],
               cp['mu']['l2']['w'], b1(cp['mu']['l2']['b']),
               cp['msg']['w'], b1(cp['msg']['b']),
               b1(cp['att_ln_g']), b1(cp['att_ln_b']),
               b1(cp['msg_ln_g']), b1(cp['msg_ln_b'])]
        msg_a = _conv_msg(qd_a, kvs_a, ef, wts, t_h)
        msg_b = _conv_msg(qd_b, kvs_b, ef, wts, t_h, ef_off_blocks=e_a // t_h)
        if _USE_SC:
            part = _sc_scatter_add(msg_a, msg_b, idx_dst_a, idx_dst_b, zeros_tab)
            return part[0], part[1]
        p = (jax.ops.segment_sum(msg_a, dst[:e_a], num_segments=np_)
             + jax.ops.segment_sum(msg_b, dst[e_a:], num_segments=np_))
        return p, None

    def upd_stage(node, p0, p1, cp):
        if p1 is None:
            p1 = jnp.zeros_like(p0)
        return _node_update(node, p0, p1, cp['cc']['w'], b1(cp['cc']['b']),
                            b1(cp['bn_g']), b1(cp['bn_b']), t_n)

    def conv(node, ef, cp):
        g4 = gather_stage(node, cp)
        p0, p1 = msg_stage(g4, ef, cp)
        return upd_stage(node, p0, p1, cp)

    ea_t = edge_attr.T
    nei_t = nei9.T
    cp0 = p['att0']
    g40 = gather_stage(node, cp0)
    ef = _rbf_edge(ea_t, p['rbf_lin']['w'], b1(p['rbf_lin']['b']), t_r)
    lens = _rbf_len(nei_t, p['rbf_lin']['w'], b1(p['rbf_lin']['b']), t_r)
    pt0, pt1 = msg_stage(g40, ef, cp0)
    angs = _rbf_ang(nei_t, ea_t, p['rbf_angle_lin']['w'],
                    b1(p['rbf_angle_lin']['b']), t_r)
    if pt1 is not None:
        bar = jax.lax.optimization_barrier((pt0, pt1) + tuple(angs))
        pt0, pt1 = bar[0], bar[1]
        angs = bar[2:5]

    ep = p['edge_upd']
    wlen_a = ep['len']['w'][:EMB]
    lembb = ep['lemb'] @ ep['len']['w'][EMB:] + ep['len']['b']
    lembb8 = jnp.zeros((8, EMB), f32).at[:3].set(lembb)
    ewts = [ep['q']['w'], b1(ep['q']['b']), ep['k']['w'], b1(ep['k']['b']),
            ep['v']['w'], b1(ep['v']['b']), wlen_a, lembb8,
            ep['ke1']['w'], b1(ep['ke1']['b']), ep['ke2']['w'], b1(ep['ke2']['b']),
            ep['ke3']['w'], b1(ep['ke3']['b']),
            ep['ve1']['w'], b1(ep['ve1']['b']), ep['ve2']['w'], b1(ep['ve2']['b']),
            ep['ve3']['w'], b1(ep['ve3']['b']),
            ep['e']['w'], b1(ep['e']['b']),
            ep['ku']['l1']['w'], b1(ep['ku']['l1']['b']),
            ep['ku']['l2']['w'], b1(ep['ku']['l2']['b']),
            ep['mu']['l1']['w'], b1(ep['mu']['l1']['b']),
            ep['mu']['l2']['w'], b1(ep['mu']['l2']['b']),
            ep['cc']['w'], b1(ep['cc']['b']),
            b1(ep['att_ln_g']), b1(ep['att_ln_b']),
            b1(ep['bn_g']), b1(ep['bn_b'])]
    node = upd_stage(node, pt0, pt1, cp0)
    cp1 = p['att1']
    g41 = gather_stage(node, cp1)
    ef = _edge_update(ef, lens, angs, ewts, t_e)
    pt0, pt1 = msg_stage(g41, ef, cp1)
    node = upd_stage(node, pt0, pt1, cp1)
    node = conv(node, ef, p['att2'])
    node = conv(node, ef, p['att3'])

    ps, cnt = _pool(node, batch2d, nb, t_n)
    at9 = jnp.swapaxes(cell, -2, -1).reshape(nb, 9)
    at16 = jnp.zeros((nb, 16), f32).at[:, :9].set(at9)
    w2 = jnp.zeros((EMB, 16), f32).at[:, :9].set(p['fc2']['w'])
    b2 = jnp.zeros((1, 16), f32).at[0, :9].set(p['fc2']['b'])
    outf = _head(ps, cnt, at16, p['fc1']['w'], b1(p['fc1']['b']), w2, b2)
    return outf[:, :9].reshape(nb, 3, 3)
